# baseline (speedup 1.0000x reference)
import functools

import jax
import jax.numpy as jnp
from jax import lax
from jax.experimental import pallas as pl
from jax.experimental.pallas import tpu as pltpu
from jax.experimental.pallas import tpu_sc as plsc

N_NODES = 10000
D = 128
NC, NS, L = 2, 16, 16
CHUNK = 128
REC = 2 * CHUNK
G = 4
PASSES = 2
BIN_ROWS = 2560
N_PAD2 = NC * PASSES * BIN_ROWS
ACC_ROWS = BIN_ROWS + 8
DRAIN_ROWS = BIN_ROWS // NS


def _sc_spmm(x, rec, valf, n_chunks):
    assert n_chunks % (2 * G) == 0

    mesh = plsc.VectorSubcoreMesh(
        core_axis_name="c", subcore_axis_name="s", num_cores=NC)

    @functools.partial(
        pl.kernel,
        out_type=jax.ShapeDtypeStruct((N_PAD2, D), jnp.float32),
        mesh=mesh,
        scratch_types=[
            [pltpu.VMEM((G * REC,), jnp.int32) for _ in range(2)],
            [pltpu.VMEM((G * CHUNK,), jnp.float32) for _ in range(2)],
            [pltpu.VMEM((CHUNK,), jnp.int32) for _ in range(2)],
            [pltpu.VMEM((CHUNK, D), jnp.float32) for _ in range(2)],
            [pltpu.VMEM((CHUNK, D), jnp.float32) for _ in range(2)],
            pltpu.VMEM((DRAIN_ROWS // 2, D), jnp.float32),
            pltpu.VMEM_SHARED((ACC_ROWS, D), jnp.float32),
            [pltpu.SemaphoreType.DMA for _ in range(2)],
            [pltpu.SemaphoreType.DMA for _ in range(2)],
            [pltpu.SemaphoreType.DMA for _ in range(2)],
        ],
    )
    def k(x_hbm, rec_hbm, val_hbm, out_hbm,
          recb, valb, dstb, grows, srows, stage_v, acc_sh, rsem, gsem, ssem):
        cid = lax.axis_index("c")
        sid = lax.axis_index("s")
        rec0 = sid * n_chunks * REC
        val0 = sid * n_chunks * CHUNK

        def rec_copies(gbase, half):
            return (
                pltpu.make_async_copy(
                    rec_hbm.at[pl.ds(rec0 + gbase * REC, G * REC)],
                    recb[half], rsem[half]),
                pltpu.make_async_copy(
                    val_hbm.at[pl.ds(val0 + gbase * CHUNK, G * CHUNK)],
                    valb[half], rsem[half]),
            )

        def one_pass(p, _):
            base_row = (PASSES * cid + p) * BIN_ROWS

            def zero_row(i, _):
                for j in range(D // L):
                    stage_v[i, pl.ds(j * L, L)] = jnp.zeros((L,), jnp.float32)
                return ()
            lax.fori_loop(0, DRAIN_ROWS // 2, zero_row, ())
            for q in range(2):
                pltpu.sync_copy(
                    stage_v,
                    acc_sh.at[pl.ds(sid * DRAIN_ROWS + q * (DRAIN_ROWS // 2),
                                    DRAIN_ROWS // 2)])
            plsc.subcore_barrier()

            pltpu.sync_copy(rec_hbm.at[pl.ds(rec0, G * REC)], recb[0])
            pltpu.sync_copy(val_hbm.at[pl.ds(val0, G * CHUNK)], valb[0])
            for c in rec_copies(G, 1):
                c.start()

            def body(h, _):
                for half in range(2):
                    gbase = (2 * h + half) * G

                    @pl.when(gbase > 0)
                    def _():
                        for c in rec_copies(gbase, half):
                            c.wait()

                    gh = [None, None]
                    sh = [None, None]
                    for u in range(2):
                        gh[u] = pltpu.async_copy(
                            x_hbm.at[recb[half].at[pl.ds(u * REC, CHUNK)]],
                            grows[u], gsem[u])

                    for u in range(G):
                        b = u % 2
                        gh[b].wait()

                        def rebase(i, _):
                            d = recb[half][
                                pl.ds(u * REC + CHUNK + i * L, L)] - base_row
                            oob = (d < 0) | (d >= BIN_ROWS)
                            dstb[b][pl.ds(i * L, L)] = jnp.where(
                                oob, BIN_ROWS, d)
                            return ()
                        lax.fori_loop(0, CHUNK // L, rebase, ())

                        if sh[b] is not None:
                            sh[b].wait()

                        def scale(g16, _):
                            vals = valb[half][pl.ds(u * CHUNK + g16 * L, L)]
                            for l in range(L):
                                e = g16 * L + l
                                v = vals[l]
                                for j in range(D // L):
                                    srows[b][e, pl.ds(j * L, L)] = (
                                        grows[b][e, pl.ds(j * L, L)] * v)
                            return ()
                        lax.fori_loop(0, CHUNK // L, scale, ())

                        sh[b] = pltpu.async_copy(
                            srows[b], acc_sh.at[dstb[b]], ssem[b], add=True)
                        if u + 2 < G:
                            gh[b] = pltpu.async_copy(
                                x_hbm.at[recb[half].at[
                                    pl.ds((u + 2) * REC, CHUNK)]],
                                grows[b], gsem[b])

                    sh[0].wait()
                    sh[1].wait()

                    @pl.when(gbase + 2 * G < n_chunks)
                    def _():
                        for c in rec_copies(gbase + 2 * G, half):
                            c.start()
                return ()
            lax.fori_loop(0, n_chunks // (2 * G), body, ())
            plsc.subcore_barrier()

            for q in range(2):
                r0 = sid * DRAIN_ROWS + q * (DRAIN_ROWS // 2)
                pltpu.sync_copy(acc_sh.at[pl.ds(r0, DRAIN_ROWS // 2)], stage_v)
                pltpu.sync_copy(
                    stage_v, out_hbm.at[pl.ds(base_row + r0, DRAIN_ROWS // 2)])
            plsc.subcore_barrier()
            return ()
        lax.fori_loop(0, PASSES, one_pass, ())

    return k(x, rec, valf)


def _tc_body(z_ref, w_ref, o_ref):
    o_ref[...] = jnp.maximum(
        jnp.dot(z_ref[...], w_ref[...], preferred_element_type=jnp.float32), 0.0)


def _tc_matmul_relu(zp, W):
    br = 400
    return pl.pallas_call(
        _tc_body,
        grid=(N_NODES // br,),
        in_specs=[
            pl.BlockSpec((br, D), lambda i: (i, 0)),
            pl.BlockSpec((D, D), lambda i: (0, 0)),
        ],
        out_specs=pl.BlockSpec((br, D), lambda i: (i, 0)),
        out_shape=jax.ShapeDtypeStruct((N_NODES, D), jnp.float32),
    )(zp, W)


def kernel(x, edge_index, edge_values, W):
    src = edge_index[0].astype(jnp.int32)
    dst = edge_index[1].astype(jnp.int32)
    val = edge_values.astype(jnp.float32)
    n_edges = src.shape[0]
    n_chunks = -(-n_edges // (NS * CHUNK))
    n_chunks = -(-n_chunks // (2 * G)) * (2 * G)
    pad = n_chunks * NS * CHUNK - n_edges
    if pad:
        src = jnp.concatenate([src, jnp.zeros((pad,), jnp.int32)])
        dst = jnp.concatenate([dst, jnp.zeros((pad,), jnp.int32)])
        val = jnp.concatenate([val, jnp.zeros((pad,), jnp.float32)])
    rec = jnp.stack([src.reshape(NS, n_chunks, CHUNK),
                     dst.reshape(NS, n_chunks, CHUNK)], axis=2).reshape(-1)
    zp = _sc_spmm(x, rec, val, n_chunks)
    return _tc_matmul_relu(zp, W)
